# Initial kernel scaffold; baseline (speedup 1.0000x reference)
import jax
import jax.numpy as jnp
from jax.experimental import pallas as pl


def kernel(element, open_flag, edge_index, order, graph_ids, steps, node_emb_el, node_emb_open, edge_embs, W1, b1, W2, b2, gamma, beta, gate_W, gate_b, pW1, pb1, pW2, pb2, pW3, pb3):
    raise NotImplementedError("write your pallas kernel here")


# speedup vs baseline: 5.7277x; 5.7277x over previous
import functools

import jax
import jax.numpy as jnp
from jax import lax
from jax.experimental import pallas as pl
from jax.experimental.pallas import tpu as pltpu
from jax.experimental.pallas import tpu_sc as plsc

NC = 2
NS = 16
K = 128
ZROWS = 16
HALF = 64


def _make_edge_pass(n_chunks, d, npad):
  rows_per_tile = npad // NS
  zgroups = rows_per_tile // ZROWS
  mesh = plsc.VectorSubcoreMesh(core_axis_name="c", subcore_axis_name="s")

  @functools.partial(
      pl.kernel,
      out_type=jax.ShapeDtypeStruct((NC, npad, d), jnp.float32),
      mesh=mesh,
      compiler_params=pltpu.CompilerParams(use_tc_tiling_on_sc=False),
      scratch_types=[
          pltpu.VMEM((n_chunks, K), jnp.int32),
          pltpu.VMEM((n_chunks, K), jnp.int32),
          pltpu.VMEM((K, d), jnp.float32),
          pltpu.VMEM((K, d), jnp.float32),
          pltpu.VMEM((ZROWS, d), jnp.float32),
          pltpu.VMEM_SHARED((npad, d), jnp.float32),
          pltpu.SemaphoreType.DMA,
          pltpu.SemaphoreType.DMA,
          pltpu.SemaphoreType.DMA,
          pltpu.SemaphoreType.DMA,
      ],
  )
  def kern(table_hbm, gidx_hbm, sidx_hbm, out_hbm,
           gidx_v, sidx_v, buf0, buf1, zbuf, agg_sh,
           gsem0, gsem1, ssem0, ssem1):
    cid = lax.axis_index("c")
    sid = lax.axis_index("s")

    pltpu.sync_copy(gidx_hbm.at[cid, sid], gidx_v)
    pltpu.sync_copy(sidx_hbm.at[cid, sid], sidx_v)

    for r in range(ZROWS):
      for c0 in range(0, d, 16):
        zbuf[r, pl.ds(c0, 16)] = jnp.zeros((16,), jnp.float32)

    base = sid * rows_per_tile

    def zinit(i, carry):
      pltpu.sync_copy(zbuf, agg_sh.at[pl.ds(base + i * ZROWS, ZROWS)])
      return carry

    lax.fori_loop(0, zgroups, zinit, 0)
    plsc.subcore_barrier()

    pltpu.async_copy(table_hbm.at[gidx_v.at[0]], buf0, gsem0)
    pltpu.async_copy(table_hbm.at[gidx_v.at[1]], buf1, gsem1)
    npairs = n_chunks // 2

    def pair(i, carry):
      j0 = 2 * i
      pltpu.make_async_copy(table_hbm.at[pl.ds(0, K)], buf0, gsem0).wait()
      pltpu.async_copy(buf0, agg_sh.at[sidx_v.at[j0]], ssem0, add=True).wait()

      @pl.when(j0 + 2 < n_chunks)
      def _():
        pltpu.async_copy(table_hbm.at[gidx_v.at[j0 + 2]], buf0, gsem0)

      pltpu.make_async_copy(table_hbm.at[pl.ds(0, K)], buf1, gsem1).wait()
      pltpu.async_copy(buf1, agg_sh.at[sidx_v.at[j0 + 1]], ssem1,
                       add=True).wait()

      @pl.when(j0 + 3 < n_chunks)
      def _():
        pltpu.async_copy(table_hbm.at[gidx_v.at[j0 + 3]], buf1, gsem1)

      return carry

    lax.fori_loop(0, npairs, pair, 0)
    plsc.subcore_barrier()
    pltpu.sync_copy(agg_sh.at[pl.ds(base, rows_per_tile)],
                    out_hbm.at[cid, pl.ds(base, rows_per_tile)])

  return kern


def _embed_body(el_ref, op_ref, embel_ref, embop_ref, h0_ref):
  n = el_ref.shape[0]
  el = el_ref[...]
  op = op_ref[...]
  h0 = jnp.zeros((n, embel_ref.shape[1]), jnp.float32)
  for t in range(embel_ref.shape[0]):
    h0 = h0 + jnp.where(el == t, 1.0, 0.0) * embel_ref[t:t + 1, :]
  for t in range(embop_ref.shape[0]):
    h0 = h0 + jnp.where(op == t, 1.0, 0.0) * embop_ref[t:t + 1, :]
  h0_ref[0] = h0[:, :HALF]
  h0_ref[1] = h0[:, HALF:]


def _tab_body(h_ref, eemb_ref, out_ref):
  out_ref[0, 0] = h_ref[0] + eemb_ref[0, 0:1, :HALF]
  out_ref[1, 0] = h_ref[1] + eemb_ref[0, 0:1, HALF:]


def _make_tab(h, edge_emb, interpret=False):
  n = h.shape[1]
  nbond, emb = edge_emb.shape
  return pl.pallas_call(
      _tab_body,
      grid=(nbond,),
      in_specs=[
          pl.BlockSpec((NC, n, HALF), lambda b: (0, 0, 0)),
          pl.BlockSpec((1, 1, emb), lambda b: (b, 0, 0)),
      ],
      out_specs=pl.BlockSpec((NC, 1, n, HALF), lambda b: (0, b, 0, 0)),
      out_shape=jax.ShapeDtypeStruct((NC, nbond, n, HALF), jnp.float32),
      interpret=interpret,
  )(h, edge_emb.reshape(nbond, 1, emb))


def _mlp_body(aggp_ref, w1_ref, b1_ref, w2_ref, b2_ref,
              g_ref, be_ref, out_ref, *, last):
  n = out_ref.shape[1]
  agg = jnp.concatenate([aggp_ref[0, :n, :], aggp_ref[1, :n, :]], axis=1)
  z1 = jnp.maximum(
      jnp.dot(agg, w1_ref[...], preferred_element_type=jnp.float32)
      + b1_ref[...], 0.0)
  z = jnp.dot(z1, w2_ref[...], preferred_element_type=jnp.float32) + b2_ref[...]
  mu = jnp.mean(z, axis=0, keepdims=True)
  zc = z - mu
  var = jnp.mean(zc * zc, axis=0, keepdims=True)
  out = zc / jnp.sqrt(var + 1e-5) * g_ref[...] + be_ref[...]
  if not last:
    out = jnp.maximum(out, 0.0)
  out_ref[0] = out[:, :HALF]
  out_ref[1] = out[:, HALF:]


def _pool_body(h_ref, gid_ref, steps_ref, gw_ref, gb_ref,
               pw1a_ref, pw1b_ref, pb1_ref, pw2_ref, pb2_ref,
               pw3_ref, pb3_ref, out_ref):
  n = h_ref.shape[1]
  nb = out_ref.shape[0]
  h = jnp.concatenate([h_ref[0], h_ref[1]], axis=1)
  gate = jnp.sum(h * gw_ref[...], axis=1, keepdims=True) + gb_ref[0, 0]
  gid = gid_ref[...]
  iob = lax.broadcasted_iota(jnp.int32, (n, nb), 1)
  m = (gid == iob).astype(jnp.float32)
  neg = jnp.float32(-1e30)
  gmax = jnp.max(jnp.where(gid == iob, gate, neg), axis=0, keepdims=True)
  gmax_n = jnp.sum(m * gmax, axis=1, keepdims=True)
  ex = jnp.exp(gate - gmax_n)
  denom = jnp.sum(m * ex, axis=0, keepdims=True)
  denom_n = jnp.sum(m * denom, axis=1, keepdims=True)
  alpha = ex / denom_n
  readout = jax.lax.dot_general(m, h * alpha, (((0,), (0,)), ((), ())),
                                preferred_element_type=jnp.float32)
  x1 = jnp.maximum(
      jnp.dot(readout, pw1a_ref[...], preferred_element_type=jnp.float32)
      + steps_ref[...] * pw1b_ref[...] + pb1_ref[...], 0.0)
  x2 = jnp.maximum(
      jnp.dot(x1, pw2_ref[...], preferred_element_type=jnp.float32)
      + pb2_ref[...], 0.0)
  out_ref[...] = (jnp.sum(x2 * pw3_ref[...], axis=1, keepdims=True)
                  + pb3_ref[...])


def _tc_call(body, out_shape, *args, interpret=False):
  return pl.pallas_call(body, out_shape=out_shape, interpret=interpret)(*args)


def _shard(x, shards, n_chunks, pad):
  e = x.shape[0]
  return jnp.concatenate([x, pad]).reshape(shards, n_chunks, K)


def kernel(element, open_flag, edge_index, order, graph_ids, steps,
           node_emb_el, node_emb_open, edge_embs, W1, b1, W2, b2,
           gamma, beta, gate_W, gate_b, pW1, pb1, pW2, pb2, pW3, pb3):
  n = element.shape[0]
  e = edge_index.shape[1]
  emb = node_emb_el.shape[1]
  n_layers = edge_embs.shape[0]
  nbond = edge_embs.shape[1]
  nb = steps.shape[0]

  npad = ((n + NS * ZROWS - 1) // (NS * ZROWS)) * (NS * ZROWS)
  if npad < n + K:
    npad += NS * ZROWS

  src = edge_index[0].astype(jnp.int32)
  dst = edge_index[1].astype(jnp.int32)
  ordr = order.astype(jnp.int32)
  perm = jnp.argsort(dst, stable=True)
  src = src[perm]
  dst = dst[perm]
  ordr = ordr[perm]

  nch = ((e + NS - 1) // NS + K - 1) // K
  nch += nch % 2
  npad_e = NS * nch * K - e
  ar = jnp.arange(npad_e, dtype=jnp.int32)
  gval = ordr * n + src
  g_pad = ar % n
  s_pad = n + ar % (npad - n)
  g16 = _shard(gval, NS, nch, g_pad)
  gidx = jnp.stack([g16, g16 + nbond * n])
  sidx = jnp.broadcast_to(_shard(dst, NS, nch, s_pad), (NC, NS, nch, K))

  edge_pass = _make_edge_pass(nch, HALF, npad)

  h = _tc_call(
      _embed_body,
      jax.ShapeDtypeStruct((NC, n, HALF), jnp.float32),
      element.astype(jnp.int32).reshape(n, 1),
      open_flag.astype(jnp.int32).reshape(n, 1),
      node_emb_el, node_emb_open)

  for l in range(n_layers):
    tab = _make_tab(h, edge_embs[l])
    aggp = edge_pass(tab.reshape(NC * nbond * n, HALF), gidx, sidx)
    h = _tc_call(
        functools.partial(_mlp_body, last=(l == n_layers - 1)),
        jax.ShapeDtypeStruct((NC, n, HALF), jnp.float32),
        aggp, W1[l], b1[l].reshape(1, -1), W2[l], b2[l].reshape(1, -1),
        gamma[l].reshape(1, -1), beta[l].reshape(1, -1))

  out = _tc_call(
      _pool_body,
      jax.ShapeDtypeStruct((nb, 1), jnp.float32),
      h, graph_ids.astype(jnp.int32).reshape(n, 1), steps,
      gate_W.reshape(1, emb), gate_b.reshape(1, 1),
      pW1[:emb], pW1[emb:].reshape(1, -1), pb1.reshape(1, -1),
      pW2, pb2.reshape(1, -1), pW3.reshape(1, -1), pb3.reshape(1, 1))
  return out
